# baseline (speedup 1.0000x reference)
import dataclasses

import jax
import jax.numpy as jnp
from jax import lax
from jax.experimental import pallas as pl
from jax.experimental.pallas import tpu as pltpu
from jax.experimental.pallas import tpu_sc as plsc

_ROWS = 4096
_COLS = 8192
_L = 16
_MINOR = 128
_SUB = _MINOR // _L
_MAJOR = _COLS // _MINOR
_MAGIC = 12582912.0


def _quant(v, width, frac_width):
    s = 2.0 ** frac_width
    th = 2.0 ** (width - 1)
    return jnp.clip(jnp.round(v * s), -th, th - 1.0) / s


def _make_tables(scale):
    addr = jnp.arange(256)
    mag = jnp.where(addr >= 128, addr - 256, addr).astype(jnp.float32)
    table_exp = _quant(jnp.exp(mag / 16.0 * scale), 8, 6)
    n = jnp.arange(1024).astype(jnp.float32)
    table_div = _quant(1.0 / n, 8, 7)
    return table_exp, table_div


def _row_body(texp_ref, tdiv_ref, idx_ref, acc_ref, x_vmem, o_vmem):
    acc_ref[...] = jnp.zeros((_L,), jnp.float32)

    @pl.loop(0, _MAJOR)
    def _pass1(j):
        for l in range(_SUB):
            v = x_vmem[0, j, pl.ds(l * _L, _L)] * 16.0
            v = jnp.minimum(jnp.maximum(v, -128.0), 127.0)
            v = (v + _MAGIC) - _MAGIC
            k = v.astype(jnp.int32) & 255
            idx_ref[j, pl.ds(l * _L, _L)] = k
            e = plsc.load_gather(texp_ref, [k])
            acc_ref[...] = acc_ref[...] + e

    s = jnp.sum(acc_ref[...])

    @pl.loop(0, _MAJOR)
    def _pass2(j):
        for l in range(_SUB):
            k = idx_ref[j, pl.ds(l * _L, _L)]
            e = plsc.load_gather(texp_ref, [k])
            y = s / e
            n = jnp.minimum(y, 1023.0).astype(jnp.int32)
            n = jnp.where(e == 0.0, 1023, n)
            o_vmem[0, j, pl.ds(l * _L, _L)] = plsc.load_gather(tdiv_ref, [n])


def _sc_qhash(x3, table_exp, table_div):
    mesh = plsc.VectorSubcoreMesh(core_axis_name="c", subcore_axis_name="s")
    cp = pltpu.CompilerParams()
    if "needs_layout_passes" in pltpu.CompilerParams.__dataclass_fields__:
        cp = dataclasses.replace(cp, needs_layout_passes=False)

    @pl.kernel(
        compiler_params=cp,
        out_type=jax.ShapeDtypeStruct((_ROWS, _MAJOR, _MINOR), jnp.float32),
        mesh=mesh,
        scratch_types=[
            pltpu.VMEM((256,), jnp.float32),
            pltpu.VMEM((1024,), jnp.float32),
            pltpu.VMEM((_MAJOR, _MINOR), jnp.int32),
            pltpu.VMEM((_L,), jnp.float32),
        ],
    )
    def _k(x_hbm, texp_hbm, tdiv_hbm, o_hbm, texp_ref, tdiv_ref, idx_ref, acc_ref):
        pltpu.sync_copy(texp_hbm, texp_ref)
        pltpu.sync_copy(tdiv_hbm, tdiv_ref)

        def body(x_vmem, o_vmem):
            _row_body(texp_ref, tdiv_ref, idx_ref, acc_ref, x_vmem, o_vmem)

        pltpu.emit_pipeline(
            body,
            grid=(_ROWS,),
            in_specs=[pl.BlockSpec((1, _MAJOR, _MINOR), lambda r: (r, 0, 0))],
            out_specs=[pl.BlockSpec((1, _MAJOR, _MINOR), lambda r: (r, 0, 0))],
            core_axis_name=("c", "s"),
            dimension_semantics=(pltpu.PARALLEL,),
        )(x_hbm, o_hbm)

    return _k(x3, table_exp, table_div)


def kernel(x, scale):
    orig_shape = x.shape
    table_exp, table_div = _make_tables(scale)
    x3 = x.reshape(_ROWS, _MAJOR, _MINOR)
    out = _sc_qhash(x3, table_exp, table_div)
    return out.reshape(orig_shape)


# speedup vs baseline: 1.1747x; 1.1747x over previous
import dataclasses

import jax
import jax.numpy as jnp
from jax import lax
from jax.experimental import pallas as pl
from jax.experimental.pallas import tpu as pltpu
from jax.experimental.pallas import tpu_sc as plsc

_ROWS = 4096
_COLS = 8192
_L = 16
_MINOR = 128
_SUB = _MINOR // _L
_MAJOR = _COLS // _MINOR
_MAGIC = 12582912.0


def _quant(v, width, frac_width):
    s = 2.0 ** frac_width
    th = 2.0 ** (width - 1)
    return jnp.clip(jnp.round(v * s), -th, th - 1.0) / s


def _make_tables(scale):
    addr = jnp.arange(256)
    mag = jnp.where(addr >= 128, addr - 256, addr).astype(jnp.float32)
    table_exp = _quant(jnp.exp(mag / 16.0 * scale), 8, 6)
    n = jnp.arange(1024).astype(jnp.float32)
    table_div = _quant(1.0 / n, 8, 7)
    return table_exp, table_div


def _row_body(texp_ref, tdiv_ref, idx_ref, x_vmem, o_vmem):
    zero = jnp.zeros((_L,), jnp.float32)

    def _pass1(j, accs):
        out = []
        for l in range(_SUB):
            v = x_vmem[0, j, pl.ds(l * _L, _L)] * 16.0
            v = jnp.minimum(jnp.maximum(v, -128.0), 127.0)
            v = (v + _MAGIC) - _MAGIC
            k = v.astype(jnp.int32) & 255
            idx_ref[j, pl.ds(l * _L, _L)] = k
            e = plsc.load_gather(texp_ref, [k])
            out.append(accs[l] + e)
        return tuple(out)

    accs = lax.fori_loop(0, _MAJOR, _pass1, (zero,) * _SUB)
    tot = accs[0]
    for l in range(1, _SUB):
        tot = tot + accs[l]
    s = jnp.sum(tot)

    @pl.loop(0, _MAJOR)
    def _pass2(j):
        for l in range(_SUB):
            k = idx_ref[j, pl.ds(l * _L, _L)]
            e = plsc.load_gather(texp_ref, [k])
            y = s / e
            n = jnp.minimum(y, 1023.0).astype(jnp.int32)
            n = jnp.where(e == 0.0, 1023, n)
            o_vmem[0, j, pl.ds(l * _L, _L)] = plsc.load_gather(tdiv_ref, [n])


def _sc_qhash(x3, table_exp, table_div):
    mesh = plsc.VectorSubcoreMesh(core_axis_name="c", subcore_axis_name="s")
    cp = pltpu.CompilerParams()
    if "needs_layout_passes" in pltpu.CompilerParams.__dataclass_fields__:
        cp = dataclasses.replace(cp, needs_layout_passes=False)
    if "use_tc_tiling_on_sc" in pltpu.CompilerParams.__dataclass_fields__:
        cp = dataclasses.replace(cp, use_tc_tiling_on_sc=True)

    @pl.kernel(
        compiler_params=cp,
        out_type=jax.ShapeDtypeStruct((_ROWS, _MAJOR, _MINOR), jnp.float32),
        mesh=mesh,
        scratch_types=[
            pltpu.VMEM((256,), jnp.float32),
            pltpu.VMEM((1024,), jnp.float32),
            pltpu.VMEM((_MAJOR, _MINOR), jnp.int32),
        ],
    )
    def _k(x_hbm, texp_hbm, tdiv_hbm, o_hbm, texp_ref, tdiv_ref, idx_ref):
        pltpu.sync_copy(texp_hbm, texp_ref)
        pltpu.sync_copy(tdiv_hbm, tdiv_ref)

        def body(x_vmem, o_vmem):
            _row_body(texp_ref, tdiv_ref, idx_ref, x_vmem, o_vmem)

        pltpu.emit_pipeline(
            body,
            grid=(_ROWS,),
            in_specs=[pl.BlockSpec((1, _MAJOR, _MINOR), lambda r: (r, 0, 0))],
            out_specs=[pl.BlockSpec((1, _MAJOR, _MINOR), lambda r: (r, 0, 0))],
            core_axis_name=("c", "s"),
            dimension_semantics=(pltpu.PARALLEL,),
        )(x_hbm, o_hbm)

    return _k(x3, table_exp, table_div)


def kernel(x, scale):
    orig_shape = x.shape
    table_exp, table_div = _make_tables(scale)
    x3 = x.reshape(_ROWS, _MAJOR, _MINOR)
    out = _sc_qhash(x3, table_exp, table_div)
    return out.reshape(orig_shape)


# speedup vs baseline: 5.8066x; 4.9430x over previous
import dataclasses

import jax
import jax.numpy as jnp
from jax import lax
from jax.experimental import pallas as pl
from jax.experimental.pallas import tpu as pltpu
from jax.experimental.pallas import tpu_sc as plsc

_ROWS = 4096
_COLS = 8192
_L = 16
_MINOR = 128
_SUB = _MINOR // _L
_MAJOR = _COLS // _MINOR
_MAGIC = 12582912.0


def _quant(v, width, frac_width):
    s = 2.0 ** frac_width
    th = 2.0 ** (width - 1)
    return jnp.clip(jnp.round(v * s), -th, th - 1.0) / s


def _make_tables(scale):
    addr = jnp.arange(256)
    mag = jnp.where(addr >= 128, addr - 256, addr).astype(jnp.float32)
    table_exp = _quant(jnp.exp(mag / 16.0 * scale), 8, 6)
    n = jnp.arange(1024).astype(jnp.float32)
    table_div = _quant(1.0 / n, 8, 7)
    return table_exp, table_div


def _row_body(texp_ref, tdiv_ref, idx_ref, flut_ref, x_vmem, o_vmem):
    zero = jnp.zeros((_L,), jnp.float32)

    def _pass1(j, accs):
        xs = [x_vmem[0, j, pl.ds(l * _L, _L)] for l in range(_SUB)]
        vs = [v * 16.0 for v in xs]
        vs = [jnp.maximum(v, -128.0) for v in vs]
        vs = [jnp.minimum(v, 127.0) for v in vs]
        vs = [v + _MAGIC for v in vs]
        ks = [lax.bitcast_convert_type(v, jnp.int32) & 255 for v in vs]
        for l in range(_SUB):
            idx_ref[j, pl.ds(l * _L, _L)] = ks[l]
        es = [plsc.load_gather(texp_ref, [k]) for k in ks]
        return tuple(accs[l] + es[l] for l in range(_SUB))

    accs = lax.fori_loop(0, _MAJOR, _pass1, (zero,) * _SUB)
    tot = accs[0]
    for l in range(1, _SUB):
        tot = tot + accs[l]
    s = jnp.sum(tot)

    for c in range(256 // _L):
        e = texp_ref[pl.ds(c * _L, _L)]
        y = s / e
        n = jnp.minimum(y, 1023.0).astype(jnp.int32)
        n = jnp.where(e == 0.0, 1023, n)
        flut_ref[pl.ds(c * _L, _L)] = plsc.load_gather(tdiv_ref, [n])

    @pl.loop(0, _MAJOR)
    def _pass2(j):
        ks = [idx_ref[j, pl.ds(l * _L, _L)] for l in range(_SUB)]
        ds_ = [plsc.load_gather(flut_ref, [k]) for k in ks]
        for l in range(_SUB):
            o_vmem[0, j, pl.ds(l * _L, _L)] = ds_[l]


def _sc_qhash(x3, table_exp, table_div):
    mesh = plsc.VectorSubcoreMesh(core_axis_name="c", subcore_axis_name="s")
    cp = pltpu.CompilerParams()
    if "needs_layout_passes" in pltpu.CompilerParams.__dataclass_fields__:
        cp = dataclasses.replace(cp, needs_layout_passes=False)
    if "use_tc_tiling_on_sc" in pltpu.CompilerParams.__dataclass_fields__:
        cp = dataclasses.replace(cp, use_tc_tiling_on_sc=True)

    @pl.kernel(
        compiler_params=cp,
        out_type=jax.ShapeDtypeStruct((_ROWS, _MAJOR, _MINOR), jnp.float32),
        mesh=mesh,
        scratch_types=[
            pltpu.VMEM((256,), jnp.float32),
            pltpu.VMEM((1024,), jnp.float32),
            pltpu.VMEM((_MAJOR, _MINOR), jnp.int32),
            pltpu.VMEM((256,), jnp.float32),
        ],
    )
    def _k(x_hbm, texp_hbm, tdiv_hbm, o_hbm, texp_ref, tdiv_ref, idx_ref, flut_ref):
        pltpu.sync_copy(texp_hbm, texp_ref)
        pltpu.sync_copy(tdiv_hbm, tdiv_ref)

        def body(x_vmem, o_vmem):
            _row_body(texp_ref, tdiv_ref, idx_ref, flut_ref, x_vmem, o_vmem)

        pltpu.emit_pipeline(
            body,
            grid=(_ROWS,),
            in_specs=[pl.BlockSpec((1, _MAJOR, _MINOR), lambda r: (r, 0, 0))],
            out_specs=[pl.BlockSpec((1, _MAJOR, _MINOR), lambda r: (r, 0, 0))],
            core_axis_name=("c", "s"),
            dimension_semantics=(pltpu.PARALLEL,),
        )(x_hbm, o_hbm)

    return _k(x3, table_exp, table_div)


def kernel(x, scale):
    orig_shape = x.shape
    table_exp, table_div = _make_tables(scale)
    x3 = x.reshape(_ROWS, _MAJOR, _MINOR)
    out = _sc_qhash(x3, table_exp, table_div)
    return out.reshape(orig_shape)
